# baseline (speedup 1.0000x reference)
import functools

import jax
import jax.numpy as jnp
import numpy as np
from jax import lax
from jax.experimental import pallas as pl
from jax.experimental.pallas import tpu as pltpu
from jax.experimental.pallas import tpu_sc as plsc

_MULT_SIZE = 1.2


def _tc_body(species_ref, coordsT_ref, coordsT_out_ref, true_atoms_ref):
    nat = species_ref.shape[0]
    pad_nat = true_atoms_ref.shape[0]
    add = pad_nat - nat
    s = species_ref[...]
    true_atoms_ref[0:nat] = s > 0
    true_atoms_ref[nat:pad_nat] = jnp.zeros((add,), jnp.bool_)
    coordsT_out_ref[:, 0:nat] = coordsT_ref[...]
    coordsT_out_ref[:, nat:pad_nat] = jnp.zeros(
        (coordsT_ref.shape[0], add), coordsT_ref.dtype)


def _make_sc_call(nat, nsys, pad_nat, dtype):
    add = pad_nat - nat
    info = plsc.get_sparse_core_info()
    nw = info.num_cores * info.num_subcores
    chunk = nat // nw
    fill_w = (add // nw) // 8 * 8
    rem = add - nw * fill_w
    rem_off = nat + nw * fill_w
    fbuf_len = max(((fill_w + 15) // 16) * 16, ((rem + 15) // 16) * 16)

    mesh = plsc.VectorSubcoreMesh(core_axis_name="c", subcore_axis_name="s")

    @functools.partial(
        pl.kernel,
        out_type=[
            jax.ShapeDtypeStruct((pad_nat,), dtype),
            jax.ShapeDtypeStruct((nsys + 1,), dtype),
            jax.ShapeDtypeStruct((pad_nat,), dtype),
        ],
        mesh=mesh,
        scratch_types=[
            pltpu.VMEM((chunk,), dtype),
            pltpu.VMEM((chunk,), dtype),
            pltpu.VMEM((fbuf_len,), dtype),
            pltpu.VMEM((32,), dtype),
        ],
    )
    def sc_pad(species_hbm, natoms_hbm, batch_hbm,
               species_out_hbm, natoms_out_hbm, batch_out_hbm,
               sbuf, bbuf, fbuf, nbuf):
        wid = lax.axis_index("s") * info.num_cores + lax.axis_index("c")
        base = wid * chunk
        pltpu.sync_copy(species_hbm.at[pl.ds(base, chunk)], sbuf)
        pltpu.sync_copy(batch_hbm.at[pl.ds(base, chunk)], bbuf)
        pltpu.sync_copy(sbuf, species_out_hbm.at[pl.ds(base, chunk)])
        pltpu.sync_copy(bbuf, batch_out_hbm.at[pl.ds(base, chunk)])
        for i in range(fbuf_len // 16):
            fbuf[pl.ds(16 * i, 16)] = jnp.full((16,), -1, dtype)
        pltpu.sync_copy(fbuf.at[pl.ds(0, fill_w)],
                        species_out_hbm.at[pl.ds(nat + wid * fill_w, fill_w)])
        @pl.when(wid == 0)
        def _():
            pltpu.sync_copy(fbuf.at[pl.ds(0, rem)],
                            species_out_hbm.at[pl.ds(rem_off, rem)])
        for i in range(fbuf_len // 16):
            fbuf[pl.ds(16 * i, 16)] = jnp.full((16,), nsys, dtype)
        pltpu.sync_copy(fbuf.at[pl.ds(0, fill_w)],
                        batch_out_hbm.at[pl.ds(nat + wid * fill_w, fill_w)])
        @pl.when(wid == 0)
        def _():
            pltpu.sync_copy(fbuf.at[pl.ds(0, rem)],
                            batch_out_hbm.at[pl.ds(rem_off, rem)])
        @pl.when(wid == 1)
        def _():
            pltpu.sync_copy(natoms_hbm, nbuf.at[pl.ds(0, nsys)])
            nbuf[pl.ds(nsys, 16)] = jnp.full((16,), add, dtype)
            pltpu.sync_copy(nbuf.at[pl.ds(0, nsys + 1)], natoms_out_hbm)

    return sc_pad


def kernel(species, natoms, batch_index, coordinates, cells):
    nat = species.shape[0]
    nsys = natoms.shape[0]
    pad_nat = int(_MULT_SIZE * nat) + 1
    ndim = coordinates.shape[1]

    sc_pad = _make_sc_call(nat, nsys, pad_nat, species.dtype)
    species_out, natoms_out, batch_out = sc_pad(species, natoms, batch_index)

    tc_out_shape = (
        jax.ShapeDtypeStruct((ndim, pad_nat), coordinates.dtype),
        jax.ShapeDtypeStruct((pad_nat,), jnp.bool_),
    )
    coordsT_out, true_atoms = pl.pallas_call(_tc_body, out_shape=tc_out_shape)(
        species, coordinates.T)

    eye = np.eye(cells.shape[1], dtype=cells.dtype)[None, :, :]
    cells_out = jnp.concatenate([cells, jnp.asarray(eye)], axis=0)
    true_sys = jnp.asarray(np.arange(nsys + 1) < nsys)
    return (species_out, natoms_out, batch_out, coordsT_out.T, cells_out,
            true_atoms, true_sys)


# speedup vs baseline: 3.7214x; 3.7214x over previous
import jax
import jax.numpy as jnp
from jax.experimental import pallas as pl

_MULT_SIZE = 1.2


def _pad_body(species_ref, natoms_ref, batch_ref, coordsT_ref,
              species_out_ref, natoms_out_ref, batch_out_ref, coordsT_out_ref,
              true_atoms_ref):
    nat = species_ref.shape[0]
    nsys = natoms_ref.shape[0]
    pad_nat = species_out_ref.shape[0]
    add = pad_nat - nat

    s = species_ref[...]
    species_out_ref[0:nat] = s
    species_out_ref[nat:pad_nat] = jnp.full((add,), -1, species_ref.dtype)
    true_atoms_ref[0:nat] = s > 0
    true_atoms_ref[nat:pad_nat] = jnp.zeros((add,), jnp.bool_)

    batch_out_ref[0:nat] = batch_ref[...]
    batch_out_ref[nat:pad_nat] = jnp.full((add,), nsys, batch_ref.dtype)

    coordsT_out_ref[:, 0:nat] = coordsT_ref[...]
    coordsT_out_ref[:, nat:pad_nat] = jnp.zeros(
        (coordsT_ref.shape[0], add), coordsT_ref.dtype)

    natoms_out_ref[0:nsys] = natoms_ref[...]
    natoms_out_ref[nsys:nsys + 1] = jnp.full((1,), add, natoms_ref.dtype)


def kernel(species, natoms, batch_index, coordinates, cells):
    nat = species.shape[0]
    nsys = natoms.shape[0]
    pad_nat = int(_MULT_SIZE * nat) + 1
    ndim = coordinates.shape[1]

    out_shape = (
        jax.ShapeDtypeStruct((pad_nat,), species.dtype),
        jax.ShapeDtypeStruct((nsys + 1,), natoms.dtype),
        jax.ShapeDtypeStruct((pad_nat,), batch_index.dtype),
        jax.ShapeDtypeStruct((ndim, pad_nat), coordinates.dtype),
        jax.ShapeDtypeStruct((pad_nat,), jnp.bool_),
    )
    (species_out, natoms_out, batch_out, coordsT_out,
     true_atoms) = pl.pallas_call(_pad_body, out_shape=out_shape)(
        species, natoms, batch_index, coordinates.T)

    cells_out = jnp.concatenate(
        [cells, jnp.eye(cells.shape[1], dtype=cells.dtype)[None, :, :]], axis=0)
    true_sys = jnp.arange(nsys + 1) < nsys
    return (species_out, natoms_out, batch_out, coordsT_out.T, cells_out,
            true_atoms, true_sys)
